# Initial kernel scaffold; baseline (speedup 1.0000x reference)
import jax
import jax.numpy as jnp
from jax.experimental import pallas as pl


def kernel(x_ncl, w1, g1, b1, w2, g2, b2):
    raise NotImplementedError("write your pallas kernel here")


# speedup vs baseline: 1.0651x; 1.0651x over previous
import jax
import jax.numpy as jnp
from jax import lax
from jax.experimental import pallas as pl
from jax.experimental.pallas import tpu as pltpu

_EPS = 1e-5


def _conv3(h, w_ref):
    L = h.shape[1]
    pos = lax.broadcasted_iota(jnp.int32, (1, L), 1)
    zero = jnp.zeros((), h.dtype)
    h_m1 = jnp.where(pos != 0, pltpu.roll(h, 1, axis=1), zero)
    h_p1 = jnp.where(pos != (L - 1), pltpu.roll(h, L - 1, axis=1), zero)
    out = jnp.dot(w_ref[0], h_m1, preferred_element_type=jnp.float32)
    out += jnp.dot(w_ref[1], h, preferred_element_type=jnp.float32)
    out += jnp.dot(w_ref[2], h_p1, preferred_element_type=jnp.float32)
    return out


def _store_conv_and_stats(out, y_ref, s_ref, ss_ref):
    y_ref[...] = out.astype(y_ref.dtype)
    s_ref[...] = jnp.sum(out, axis=1, keepdims=True)
    ss_ref[...] = jnp.sum(out * out, axis=1, keepdims=True)


def _phase1_kernel(x_ref, w_ref, y_ref, s_ref, ss_ref):
    out = _conv3(x_ref[...].astype(jnp.bfloat16), w_ref)
    _store_conv_and_stats(out, y_ref, s_ref, ss_ref)


def _phase2_kernel(y1_ref, scale_ref, shift_ref, w_ref, y_ref, s_ref, ss_ref):
    h = jnp.maximum(
        y1_ref[...].astype(jnp.float32) * scale_ref[...] + shift_ref[...], 0.0)
    out = _conv3(h.astype(jnp.bfloat16), w_ref)
    _store_conv_and_stats(out, y_ref, s_ref, ss_ref)


def _phase3_kernel(y2_ref, x_ref, scale_ref, shift_ref, o_ref):
    o_ref[...] = jnp.maximum(
        y2_ref[...].astype(jnp.float32) * scale_ref[...] + shift_ref[...]
        + x_ref[...], 0.0)


def _fold_bn(s_part, ss_part, gamma, beta, count, eps):
    s = jnp.sum(s_part[:, :, 0], axis=0)
    ss = jnp.sum(ss_part[:, :, 0], axis=0)
    mean = s / count
    var = ss / count - mean * mean
    scale = gamma.astype(jnp.float32) * lax.rsqrt(var + eps)
    shift = beta.astype(jnp.float32) - mean * scale
    p = scale.shape[0]
    return scale.reshape(p, 1), shift.reshape(p, 1)


def kernel(x_ncl, w1, g1, b1, w2, g2, b2):
    N, C, L = x_ncl.shape
    P = w1.shape[1]
    cnt = float(N * L)

    x_f = x_ncl.astype(jnp.float32)
    w1b = w1.astype(jnp.bfloat16)
    w2b = w2.astype(jnp.bfloat16)

    cparams = pltpu.CompilerParams(
        dimension_semantics=("parallel",),
        vmem_limit_bytes=32 * 1024 * 1024)

    def tile_spec(c):
        return pl.BlockSpec((None, c, L), lambda n: (n, 0, 0))

    def w_spec(c_in):
        return pl.BlockSpec((3, P, c_in), lambda n: (0, 0, 0))

    stat_spec = pl.BlockSpec((None, P, 1), lambda n: (n, 0, 0))
    vec_spec = pl.BlockSpec((P, 1), lambda n: (0, 0))

    conv_out_shape = [
        jax.ShapeDtypeStruct((N, P, L), jnp.bfloat16),
        jax.ShapeDtypeStruct((N, P, 1), jnp.float32),
        jax.ShapeDtypeStruct((N, P, 1), jnp.float32),
    ]

    y1, s1, ss1 = pl.pallas_call(
        _phase1_kernel,
        grid=(N,),
        in_specs=[tile_spec(C), w_spec(C)],
        out_specs=[tile_spec(P), stat_spec, stat_spec],
        out_shape=conv_out_shape,
        compiler_params=cparams,
    )(x_f, w1b)

    scale1, shift1 = _fold_bn(s1, ss1, g1, b1, cnt, _EPS)

    y2, s2, ss2 = pl.pallas_call(
        _phase2_kernel,
        grid=(N,),
        in_specs=[tile_spec(P), vec_spec, vec_spec, w_spec(P)],
        out_specs=[tile_spec(P), stat_spec, stat_spec],
        out_shape=conv_out_shape,
        compiler_params=cparams,
    )(y1, scale1, shift1, w2b)

    scale2, shift2 = _fold_bn(s2, ss2, g2, b2, cnt, _EPS)

    out = pl.pallas_call(
        _phase3_kernel,
        grid=(N,),
        in_specs=[tile_spec(P), tile_spec(C), vec_spec, vec_spec],
        out_specs=tile_spec(P),
        out_shape=jax.ShapeDtypeStruct((N, P, L), jnp.float32),
        compiler_params=cparams,
    )(y2, x_f, scale2, shift2)

    return out


# speedup vs baseline: 1.4597x; 1.3705x over previous
import jax
import jax.numpy as jnp
from jax import lax
from jax.experimental import pallas as pl
from jax.experimental.pallas import tpu as pltpu

_EPS = 1e-5


def _conv3(h, w_ref):
    L = h.shape[1]
    pos = lax.broadcasted_iota(jnp.int32, (1, L), 1)
    zero = jnp.zeros((), h.dtype)
    h_m1 = jnp.where(pos != 0, pltpu.roll(h, 1, axis=1), zero)
    h_p1 = jnp.where(pos != (L - 1), pltpu.roll(h, L - 1, axis=1), zero)
    out = jnp.dot(w_ref[0], h_m1, preferred_element_type=jnp.float32)
    out += jnp.dot(w_ref[1], h, preferred_element_type=jnp.float32)
    out += jnp.dot(w_ref[2], h_p1, preferred_element_type=jnp.float32)
    return out


def _accum_stats(out, n, s_ref, ss_ref):
    s = jnp.sum(out, axis=1, keepdims=True)
    ss = jnp.sum(out * out, axis=1, keepdims=True)

    @pl.when(n == 0)
    def _():
        s_ref[...] = s
        ss_ref[...] = ss

    @pl.when(n != 0)
    def _():
        s_ref[...] += s
        ss_ref[...] += ss


def _fold(s_ref, ss_ref, g_ref, b_ref, sc_ref, sh_ref, count, eps):
    mean = s_ref[...] / count
    var = ss_ref[...] / count - mean * mean
    scale = g_ref[...] * lax.rsqrt(var + eps)
    sc_ref[...] = scale
    sh_ref[...] = b_ref[...] - mean * scale


def _make_fused_kernel(N, L, eps):
    cnt = float(N * L)

    def body(x_ref, g1_ref, b1_ref, g2_ref, b2_ref, w1_ref, w2_ref,
             o_ref,
             xb_s, y1_s, y2_s, s_s, ss_s, s2_s, ss2_s,
             sc1_s, sh1_s, sc2_s, sh2_s):
        p = pl.program_id(0)
        n = pl.program_id(1)

        @pl.when(p == 0)
        def _phase1():
            xb = x_ref[...].astype(jnp.bfloat16)
            xb_s[n] = xb
            out = _conv3(xb, w1_ref)
            y1_s[n] = out.astype(jnp.bfloat16)
            _accum_stats(out, n, s_s, ss_s)

        @pl.when(jnp.logical_and(p == 1, n == 0))
        def _fold1():
            _fold(s_s, ss_s, g1_ref, b1_ref, sc1_s, sh1_s, cnt, eps)

        @pl.when(p == 1)
        def _phase2():
            h = jnp.maximum(
                y1_s[n].astype(jnp.float32) * sc1_s[...] + sh1_s[...], 0.0)
            out = _conv3(h.astype(jnp.bfloat16), w2_ref)
            y2_s[n] = out.astype(jnp.bfloat16)
            _accum_stats(out, n, s2_s, ss2_s)

        @pl.when(jnp.logical_and(p == 2, n == 0))
        def _fold2():
            _fold(s2_s, ss2_s, g2_ref, b2_ref, sc2_s, sh2_s, cnt, eps)

        @pl.when(p == 2)
        def _phase3():
            o_ref[...] = jnp.maximum(
                y2_s[n].astype(jnp.float32) * sc2_s[...] + sh2_s[...]
                + xb_s[n].astype(jnp.float32), 0.0)

    return body


def kernel(x_ncl, w1, g1, b1, w2, g2, b2):
    N, C, L = x_ncl.shape
    P = w1.shape[1]

    x_f = x_ncl.astype(jnp.float32)
    w1b = w1.astype(jnp.bfloat16)
    w2b = w2.astype(jnp.bfloat16)
    g1c = g1.astype(jnp.float32).reshape(P, 1)
    b1c = b1.astype(jnp.float32).reshape(P, 1)
    g2c = g2.astype(jnp.float32).reshape(P, 1)
    b2c = b2.astype(jnp.float32).reshape(P, 1)

    cparams = pltpu.CompilerParams(
        dimension_semantics=("arbitrary", "arbitrary"),
        vmem_limit_bytes=60 * 1024 * 1024)

    x_spec = pl.BlockSpec(
        (None, C, L), lambda p, n: (jnp.where(p == 0, n, 0), 0, 0))
    o_spec = pl.BlockSpec(
        (None, P, L), lambda p, n: (jnp.where(p == 2, n, 0), 0, 0))
    vec_spec = pl.BlockSpec((P, 1), lambda p, n: (0, 0))

    def w_spec(c_in):
        return pl.BlockSpec((3, P, c_in), lambda p, n: (0, 0, 0))

    stat = pltpu.VMEM((P, 1), jnp.float32)

    out = pl.pallas_call(
        _make_fused_kernel(N, L, _EPS),
        grid=(3, N),
        in_specs=[x_spec, vec_spec, vec_spec, vec_spec, vec_spec,
                  w_spec(C), w_spec(P)],
        out_specs=o_spec,
        out_shape=jax.ShapeDtypeStruct((N, P, L), jnp.float32),
        scratch_shapes=[
            pltpu.VMEM((N, C, L), jnp.bfloat16),
            pltpu.VMEM((N, P, L), jnp.bfloat16),
            pltpu.VMEM((N, P, L), jnp.bfloat16),
            stat, stat, stat, stat,
            stat, stat, stat, stat,
        ],
        compiler_params=cparams,
    )(x_f, g1c, b1c, g2c, b2c, w1b, w2b)

    return out


# speedup vs baseline: 1.7343x; 1.1881x over previous
import jax
import jax.numpy as jnp
from jax import lax
from jax.experimental import pallas as pl
from jax.experimental.pallas import tpu as pltpu

_EPS = 1e-5


def _conv3(h, w_ref):
    L = h.shape[1]
    pos = lax.broadcasted_iota(jnp.int32, (1, L), 1)
    zero = jnp.zeros((), h.dtype)
    h_m1 = jnp.where(pos != 0, pltpu.roll(h, 1, axis=1), zero)
    h_p1 = jnp.where(pos != (L - 1), pltpu.roll(h, L - 1, axis=1), zero)
    out = jnp.dot(w_ref[0], h_m1, preferred_element_type=jnp.float32)
    out += jnp.dot(w_ref[1], h, preferred_element_type=jnp.float32)
    out += jnp.dot(w_ref[2], h_p1, preferred_element_type=jnp.float32)
    return out


def _accum_stats(out, n, s_ref, ss_ref):
    s = jnp.sum(out, axis=1, keepdims=True)
    ss = jnp.sum(out * out, axis=1, keepdims=True)

    @pl.when(n == 0)
    def _():
        s_ref[...] = s
        ss_ref[...] = ss

    @pl.when(n != 0)
    def _():
        s_ref[...] += s
        ss_ref[...] += ss


def _fold(s_ref, ss_ref, g_ref, b_ref, sc_ref, sh_ref, count, eps):
    mean = s_ref[...] / count
    var = ss_ref[...] / count - mean * mean
    scale = g_ref[...] * lax.rsqrt(var + eps)
    sc_ref[...] = scale
    sh_ref[...] = b_ref[...] - mean * scale


def _make_fused_kernel(N, L, B, eps):
    cnt = float(N * L)

    def body(x_ref, g1_ref, b1_ref, g2_ref, b2_ref, w1_ref, w2_ref,
             o_ref,
             xb_s, y1_s, y2_s, s_s, ss_s, s2_s, ss2_s,
             sc1_s, sh1_s, sc2_s, sh2_s):
        p = pl.program_id(0)
        t = pl.program_id(1)

        @pl.when(p == 0)
        def _phase1():
            for j in range(B):
                n = t * B + j
                xb = x_ref[j].astype(jnp.bfloat16)
                xb_s[n] = xb
                out = _conv3(xb, w1_ref)
                y1_s[n] = out.astype(jnp.bfloat16)
                _accum_stats(out, n, s_s, ss_s)

        @pl.when(jnp.logical_and(p == 1, t == 0))
        def _fold1():
            _fold(s_s, ss_s, g1_ref, b1_ref, sc1_s, sh1_s, cnt, eps)

        @pl.when(p == 1)
        def _phase2():
            sc = sc1_s[...].astype(jnp.bfloat16)
            sh = sh1_s[...].astype(jnp.bfloat16)
            zero = jnp.zeros((), jnp.bfloat16)
            for j in range(B):
                n = t * B + j
                h = jnp.maximum(y1_s[n] * sc + sh, zero)
                out = _conv3(h, w2_ref)
                y2_s[n] = out.astype(jnp.bfloat16)
                _accum_stats(out, n, s2_s, ss2_s)

        @pl.when(jnp.logical_and(p == 2, t == 0))
        def _fold2():
            _fold(s2_s, ss2_s, g2_ref, b2_ref, sc2_s, sh2_s, cnt, eps)

        @pl.when(p == 2)
        def _phase3():
            for j in range(B):
                n = t * B + j
                o_ref[j] = jnp.maximum(
                    y2_s[n].astype(jnp.float32) * sc2_s[...] + sh2_s[...]
                    + xb_s[n].astype(jnp.float32), 0.0)

    return body


def kernel(x_ncl, w1, g1, b1, w2, g2, b2):
    N, C, L = x_ncl.shape
    P = w1.shape[1]

    x_f = x_ncl.astype(jnp.float32)
    w1b = w1.astype(jnp.bfloat16)
    w2b = w2.astype(jnp.bfloat16)
    g1c = g1.astype(jnp.float32).reshape(P, 1)
    b1c = b1.astype(jnp.float32).reshape(P, 1)
    g2c = g2.astype(jnp.float32).reshape(P, 1)
    b2c = b2.astype(jnp.float32).reshape(P, 1)

    B = 2

    cparams = pltpu.CompilerParams(
        dimension_semantics=("arbitrary", "arbitrary"),
        vmem_limit_bytes=60 * 1024 * 1024)

    x_spec = pl.BlockSpec(
        (B, C, L), lambda p, t: (jnp.where(p == 0, t, 0), 0, 0))
    o_spec = pl.BlockSpec(
        (B, P, L), lambda p, t: (jnp.where(p == 2, t, 0), 0, 0))
    vec_spec = pl.BlockSpec((P, 1), lambda p, t: (0, 0))

    def w_spec(c_in):
        return pl.BlockSpec((3, P, c_in), lambda p, t: (0, 0, 0))

    stat = pltpu.VMEM((P, 1), jnp.float32)

    out = pl.pallas_call(
        _make_fused_kernel(N, L, B, _EPS),
        grid=(3, N // B),
        in_specs=[x_spec, vec_spec, vec_spec, vec_spec, vec_spec,
                  w_spec(C), w_spec(P)],
        out_specs=o_spec,
        out_shape=jax.ShapeDtypeStruct((N, P, L), jnp.float32),
        scratch_shapes=[
            pltpu.VMEM((N, C, L), jnp.bfloat16),
            pltpu.VMEM((N, P, L), jnp.bfloat16),
            pltpu.VMEM((N, P, L), jnp.bfloat16),
            stat, stat, stat, stat,
            stat, stat, stat, stat,
        ],
        compiler_params=cparams,
    )(x_f, g1c, b1c, g2c, b2c, w1b, w2b)

    return out


# speedup vs baseline: 1.8682x; 1.0772x over previous
import jax
import jax.numpy as jnp
from jax import lax
from jax.experimental import pallas as pl
from jax.experimental.pallas import tpu as pltpu

_EPS = 1e-5


def _conv3(h, w_ref):
    L = h.shape[1]
    pos = lax.broadcasted_iota(jnp.int32, (1, L), 1)
    zero = jnp.zeros((), h.dtype)
    h_m1 = jnp.where(pos != 0, pltpu.roll(h, 1, axis=1), zero)
    h_p1 = jnp.where(pos != (L - 1), pltpu.roll(h, L - 1, axis=1), zero)
    out = jnp.dot(w_ref[0], h_m1, preferred_element_type=jnp.float32)
    out += jnp.dot(w_ref[1], h, preferred_element_type=jnp.float32)
    out += jnp.dot(w_ref[2], h_p1, preferred_element_type=jnp.float32)
    return out


def _accum_stats(out, n, s_ref, ss_ref):
    s = jnp.sum(out, axis=1, keepdims=True)
    ss = jnp.sum(out * out, axis=1, keepdims=True)

    @pl.when(n == 0)
    def _():
        s_ref[...] = s
        ss_ref[...] = ss

    @pl.when(n != 0)
    def _():
        s_ref[...] += s
        ss_ref[...] += ss


def _fold(s_ref, ss_ref, g_ref, b_ref, sc_ref, sh_ref, count, eps):
    mean = s_ref[...] / count
    var = ss_ref[...] / count - mean * mean
    scale = g_ref[...] * lax.rsqrt(var + eps)
    sc_ref[...] = scale
    sh_ref[...] = b_ref[...] - mean * scale


def _make_fused_kernel(N, L, B, eps):
    cnt = float(N * L)

    def body(x_ref, g1_ref, b1_ref, g2_ref, b2_ref, w1_ref, w2_ref,
             o_ref,
             xb_s, y1_s, y2_s, s_s, ss_s, s2_s, ss2_s,
             sc1_s, sh1_s, sc2_s, sh2_s):
        p = pl.program_id(0)
        t = pl.program_id(1)

        @pl.when(p == 0)
        def _phase1():
            for j in range(B):
                n = t * B + j
                xb = x_ref[j].astype(jnp.bfloat16)
                xb_s[n] = xb
                out = _conv3(xb, w1_ref)
                y1_s[n] = out.astype(jnp.bfloat16)
                _accum_stats(out, n, s_s, ss_s)

        @pl.when(jnp.logical_and(p == 1, t == 0))
        def _fold1():
            _fold(s_s, ss_s, g1_ref, b1_ref, sc1_s, sh1_s, cnt, eps)

        @pl.when(p == 1)
        def _phase2():
            sc = sc1_s[...].astype(jnp.bfloat16)
            sh = sh1_s[...].astype(jnp.bfloat16)
            zero = jnp.zeros((), jnp.bfloat16)
            for j in range(B):
                n = t * B + j
                h = jnp.maximum(y1_s[n] * sc + sh, zero)
                out = _conv3(h, w2_ref)
                y2_s[n] = out.astype(jnp.bfloat16)
                _accum_stats(out, n, s2_s, ss2_s)

        @pl.when(jnp.logical_and(p == 2, t == 0))
        def _fold2():
            _fold(s2_s, ss2_s, g2_ref, b2_ref, sc2_s, sh2_s, cnt, eps)

        @pl.when(p == 2)
        def _phase3():
            for j in range(B):
                n = t * B + j
                o_ref[j] = jnp.maximum(
                    y2_s[n].astype(jnp.float32) * sc2_s[...] + sh2_s[...]
                    + xb_s[n].astype(jnp.float32), 0.0)

    return body


def kernel(x_ncl, w1, g1, b1, w2, g2, b2):
    N, C, L = x_ncl.shape
    P = w1.shape[1]

    x_f = x_ncl.astype(jnp.float32)
    w1b = w1.astype(jnp.bfloat16)
    w2b = w2.astype(jnp.bfloat16)
    g1c = g1.astype(jnp.float32).reshape(P, 1)
    b1c = b1.astype(jnp.float32).reshape(P, 1)
    g2c = g2.astype(jnp.float32).reshape(P, 1)
    b2c = b2.astype(jnp.float32).reshape(P, 1)

    B = 4

    cparams = pltpu.CompilerParams(
        dimension_semantics=("arbitrary", "arbitrary"),
        vmem_limit_bytes=60 * 1024 * 1024)

    x_spec = pl.BlockSpec(
        (B, C, L), lambda p, t: (jnp.where(p == 0, t, 0), 0, 0))
    o_spec = pl.BlockSpec(
        (B, P, L), lambda p, t: (jnp.where(p == 2, t, 0), 0, 0))
    vec_spec = pl.BlockSpec((P, 1), lambda p, t: (0, 0))

    def w_spec(c_in):
        return pl.BlockSpec((3, P, c_in), lambda p, t: (0, 0, 0))

    stat = pltpu.VMEM((P, 1), jnp.float32)

    out = pl.pallas_call(
        _make_fused_kernel(N, L, B, _EPS),
        grid=(3, N // B),
        in_specs=[x_spec, vec_spec, vec_spec, vec_spec, vec_spec,
                  w_spec(C), w_spec(P)],
        out_specs=o_spec,
        out_shape=jax.ShapeDtypeStruct((N, P, L), jnp.float32),
        scratch_shapes=[
            pltpu.VMEM((N, C, L), jnp.bfloat16),
            pltpu.VMEM((N, P, L), jnp.bfloat16),
            pltpu.VMEM((N, P, L), jnp.bfloat16),
            stat, stat, stat, stat,
            stat, stat, stat, stat,
        ],
        compiler_params=cparams,
    )(x_f, g1c, b1c, g2c, b2c, w1b, w2b)

    return out


# speedup vs baseline: 1.9290x; 1.0325x over previous
import jax
import jax.numpy as jnp
from jax import lax
from jax.experimental import pallas as pl
from jax.experimental.pallas import tpu as pltpu

_EPS = 1e-5


def _conv3(h, w_ref):
    L = h.shape[1]
    pos = lax.broadcasted_iota(jnp.int32, (1, L), 1)
    zero = jnp.zeros((), h.dtype)
    h_m1 = jnp.where(pos != 0, pltpu.roll(h, 1, axis=1), zero)
    h_p1 = jnp.where(pos != (L - 1), pltpu.roll(h, L - 1, axis=1), zero)
    out = jnp.dot(w_ref[0], h_m1, preferred_element_type=jnp.float32)
    out += jnp.dot(w_ref[1], h, preferred_element_type=jnp.float32)
    out += jnp.dot(w_ref[2], h_p1, preferred_element_type=jnp.float32)
    return out


def _accum_stats(out, n, s_ref, ss_ref):
    s = jnp.sum(out, axis=1, keepdims=True)
    ss = jnp.sum(out * out, axis=1, keepdims=True)

    @pl.when(n == 0)
    def _():
        s_ref[...] = s
        ss_ref[...] = ss

    @pl.when(n != 0)
    def _():
        s_ref[...] += s
        ss_ref[...] += ss


def _fold(s_ref, ss_ref, g_ref, b_ref, sc_ref, sh_ref, count, eps):
    mean = s_ref[...] / count
    var = ss_ref[...] / count - mean * mean
    scale = g_ref[...] * lax.rsqrt(var + eps)
    sc_ref[...] = scale
    sh_ref[...] = b_ref[...] - mean * scale


def _make_fused_kernel(N, L, B, eps):
    cnt = float(N * L)

    def body(x_ref, g1_ref, b1_ref, g2_ref, b2_ref, w1_ref, w2_ref,
             o_ref,
             xb_s, y_s, s_s, ss_s, s2_s, ss2_s,
             sc1_s, sh1_s, sc2_s, sh2_s):
        p = pl.program_id(0)
        t = pl.program_id(1)

        @pl.when(p == 0)
        def _phase1():
            for j in range(B):
                n = t * B + j
                xb = x_ref[j].astype(jnp.bfloat16)
                xb_s[n] = xb
                out = _conv3(xb, w1_ref)
                y_s[n] = out.astype(jnp.bfloat16)
                _accum_stats(out, n, s_s, ss_s)

        @pl.when(jnp.logical_and(p == 1, t == 0))
        def _fold1():
            _fold(s_s, ss_s, g1_ref, b1_ref, sc1_s, sh1_s, cnt, eps)

        @pl.when(p == 1)
        def _phase2():
            sc = sc1_s[...].astype(jnp.bfloat16)
            sh = sh1_s[...].astype(jnp.bfloat16)
            zero = jnp.zeros((), jnp.bfloat16)
            for j in range(B):
                n = t * B + j
                h = jnp.maximum(y_s[n] * sc + sh, zero)
                out = _conv3(h, w2_ref)
                y_s[n] = out.astype(jnp.bfloat16)
                _accum_stats(out, n, s2_s, ss2_s)

        @pl.when(jnp.logical_and(p == 2, t == 0))
        def _fold2():
            _fold(s2_s, ss2_s, g2_ref, b2_ref, sc2_s, sh2_s, cnt, eps)

        @pl.when(p == 2)
        def _phase3():
            for j in range(B):
                n = t * B + j
                o_ref[j] = jnp.maximum(
                    y_s[n].astype(jnp.float32) * sc2_s[...] + sh2_s[...]
                    + xb_s[n].astype(jnp.float32), 0.0)

    return body


def kernel(x_ncl, w1, g1, b1, w2, g2, b2):
    N, C, L = x_ncl.shape
    P = w1.shape[1]

    x_f = x_ncl.astype(jnp.float32)
    w1b = w1.astype(jnp.bfloat16)
    w2b = w2.astype(jnp.bfloat16)
    g1c = g1.astype(jnp.float32).reshape(P, 1)
    b1c = b1.astype(jnp.float32).reshape(P, 1)
    g2c = g2.astype(jnp.float32).reshape(P, 1)
    b2c = b2.astype(jnp.float32).reshape(P, 1)

    B = 8

    cparams = pltpu.CompilerParams(
        dimension_semantics=("arbitrary", "arbitrary"),
        vmem_limit_bytes=60 * 1024 * 1024)

    x_spec = pl.BlockSpec(
        (B, C, L), lambda p, t: (jnp.where(p == 0, t, 0), 0, 0))
    o_spec = pl.BlockSpec(
        (B, P, L), lambda p, t: (jnp.where(p == 2, t, 0), 0, 0))
    vec_spec = pl.BlockSpec((P, 1), lambda p, t: (0, 0))

    def w_spec(c_in):
        return pl.BlockSpec((3, P, c_in), lambda p, t: (0, 0, 0))

    stat = pltpu.VMEM((P, 1), jnp.float32)

    out = pl.pallas_call(
        _make_fused_kernel(N, L, B, _EPS),
        grid=(3, N // B),
        in_specs=[x_spec, vec_spec, vec_spec, vec_spec, vec_spec,
                  w_spec(C), w_spec(P)],
        out_specs=o_spec,
        out_shape=jax.ShapeDtypeStruct((N, P, L), jnp.float32),
        scratch_shapes=[
            pltpu.VMEM((N, C, L), jnp.bfloat16),
            pltpu.VMEM((N, P, L), jnp.bfloat16),
            stat, stat, stat, stat,
            stat, stat, stat, stat,
        ],
        compiler_params=cparams,
    )(x_f, g1c, b1c, g2c, b2c, w1b, w2b)

    return out
